# baseline (speedup 1.0000x reference)
import functools
import math

import jax
import jax.numpy as jnp
from jax import lax
from jax.experimental import pallas as pl
from jax.experimental.pallas import tpu as pltpu
from jax.experimental.pallas import tpu_sc as plsc

NUM_EMB = 1000000
DIM = 32
PI = 0.25
S1 = 1.0
S2 = math.exp(-6.0)

_C0 = -0.5 * math.log(2.0 * math.pi)
_K1 = math.log(PI) + _C0 - math.log(S1)
_K3_1 = 1.0 / (2.0 * S1 * S1)
_K2 = math.log(1.0 - PI) + _C0 - math.log(S2)
_K3_2 = 1.0 / (2.0 * S2 * S2)
_KL_CONST = float(NUM_EMB * DIM * _C0)

import numpy as _np


def _np_threefry2x32(k0, k1, x0, x1):
    def rotl(x, r):
        return ((x << _np.uint32(r)) | (x >> _np.uint32(32 - r))).astype(_np.uint32)

    ks0 = _np.uint32(k0)
    ks1 = _np.uint32(k1)
    ks2 = _np.uint32(ks0 ^ ks1 ^ _np.uint32(0x1BD11BDA))
    ks = [ks0, ks1, ks2]
    rots = [[13, 15, 26, 6], [17, 29, 16, 24]]
    x0 = (x0 + ks0).astype(_np.uint32)
    x1 = (x1 + ks1).astype(_np.uint32)
    for i in range(5):
        for r in rots[i % 2]:
            x0 = (x0 + x1).astype(_np.uint32)
            x1 = rotl(x1, r)
            x1 = (x1 ^ x0).astype(_np.uint32)
        x0 = (x0 + ks[(i + 1) % 3]).astype(_np.uint32)
        x1 = (x1 + ks[(i + 2) % 3] + _np.uint32(i + 1)).astype(_np.uint32)
    return x0, x1


def _np_erfinv32(x):
    x = x.astype(_np.float32)
    w = (-_np.log1p((-x * x).astype(_np.float32))).astype(_np.float32)
    small = w < _np.float32(5.0)
    ws = (w - _np.float32(2.5)).astype(_np.float32)
    wl = (_np.sqrt(_np.maximum(w, _np.float32(5.0))) - _np.float32(3.0)).astype(
        _np.float32
    )
    cs = [2.81022636e-08, 3.43273939e-07, -3.5233877e-06, -4.39150654e-06,
          0.00021858087, -0.00125372503, -0.00417768164, 0.246640727, 1.50140941]
    cl = [-0.000200214257, 0.000100950558, 0.00134934322, -0.00367342844,
          0.00573950773, -0.0076224613, 0.00943887047, 1.00167406, 2.83297682]
    ps = _np.float32(cs[0])
    for c in cs[1:]:
        ps = (_np.float32(c) + ps * ws).astype(_np.float32)
    pw = _np.float32(cl[0])
    for c in cl[1:]:
        pw = (_np.float32(c) + pw * wl).astype(_np.float32)
    p = _np.where(small, ps, pw)
    return (p * x).astype(_np.float32)


def _np_normal_key42(n):
    i = _np.arange(n, dtype=_np.uint64)
    b0, b1 = _np_threefry2x32(
        0, 42,
        (i >> _np.uint64(32)).astype(_np.uint32),
        (i & _np.uint64(0xFFFFFFFF)).astype(_np.uint32),
    )
    bits = (b0 ^ b1).astype(_np.uint32)
    f = ((bits >> _np.uint32(9)) | _np.uint32(0x3F800000)).view(_np.float32)
    u01 = (f - _np.float32(1.0)).astype(_np.float32)
    lo = _np.nextafter(_np.float32(-1.0), _np.float32(0.0), dtype=_np.float32)
    hi = _np.float32(1.0)
    u = _np.maximum(lo, (u01 * (hi - lo) + lo).astype(_np.float32))
    return (_np.float32(_np.sqrt(2.0)) * _np_erfinv32(u)).astype(_np.float32)


_EPS_T = _np.ascontiguousarray(
    _np_normal_key42(NUM_EMB * DIM).reshape(NUM_EMB, DIM).T
)

_BLKC = 8192
_GRID = -(-NUM_EMB // _BLKC)


def _tc_body(mu_ref, rho_ref, eps_ref, w_ref, kl_ref, acc_ref):
    i = pl.program_id(0)
    mu = mu_ref[...]
    rho = rho_ref[...]
    eps = eps_ref[...]
    sig = jnp.log1p(jnp.exp(rho)) + 1e-5
    w = mu + eps * sig
    wt = w.T.reshape(_BLKC // 4, 4, DIM)
    w_ref[...] = jnp.concatenate([wt[:, 0], wt[:, 1], wt[:, 2], wt[:, 3]], axis=1)
    t = w * w
    lp1 = _K1 - _K3_1 * t
    lp2 = _K2 - _K3_2 * t
    m = jnp.maximum(lp1, lp2)
    log_prior = m + jnp.log1p(jnp.exp(-jnp.abs(lp1 - lp2)))
    term = -jnp.log(sig) - 0.5 * (eps * eps) - log_prior
    col = i * _BLKC + lax.broadcasted_iota(jnp.int32, (DIM, _BLKC), 1)
    term = jnp.where(col < NUM_EMB, term, 0.0)
    part = jnp.sum(term.reshape(4, 8, _BLKC), axis=0)

    @pl.when(i == 0)
    def _init():
        acc_ref[...] = part

    @pl.when(i != 0)
    def _acc():
        acc_ref[...] = acc_ref[...] + part

    @pl.when(i == _GRID - 1)
    def _fin():
        kl_ref[0, 0] = _KL_CONST + jnp.sum(acc_ref[...])


_tc_pass = pl.pallas_call(
    _tc_body,
    grid=(_GRID,),
    in_specs=[
        pl.BlockSpec((DIM, _BLKC), lambda i: (0, i)),
        pl.BlockSpec((DIM, _BLKC), lambda i: (0, i)),
        pl.BlockSpec((DIM, _BLKC), lambda i: (0, i)),
    ],
    out_specs=[
        pl.BlockSpec((_BLKC // 4, 128), lambda i: (i, 0)),
        pl.BlockSpec((1, 1), lambda i: (0, 0), memory_space=pltpu.SMEM),
    ],
    out_shape=[
        jax.ShapeDtypeStruct((NUM_EMB * DIM // 128, 128), jnp.float32),
        jax.ShapeDtypeStruct((1, 1), jnp.float32),
    ],
    scratch_shapes=[pltpu.VMEM((8, _BLKC), jnp.float32)],
    compiler_params=pltpu.CompilerParams(
        dimension_semantics=("arbitrary",),
    ),
)

_INFO = plsc.get_sparse_core_info()
_NC = _INFO.num_cores
_NW = _INFO.num_cores * _INFO.num_subcores
_B = 16384 * 50
_NB = 16384
_NS = 50
_GRP = 128
_GPC = 8
_CH = _GRP * _GPC
_NCHT = _NS * (_NB // _CH)
_NCH = _NCHT // _NW

_sc_mesh = plsc.VectorSubcoreMesh(core_axis_name="c", subcore_axis_name="s")


@functools.partial(
    pl.kernel,
    mesh=_sc_mesh,
    out_type=jax.ShapeDtypeStruct((_NS, DIM // 8, (_NB // 128) * 8 * 128), jnp.float32),
    scratch_types=[
        pltpu.VMEM((_GPC, _GRP), jnp.int32),
        pltpu.VMEM((_CH, DIM), jnp.float32),
        pltpu.VMEM(((DIM // 8) * _GPC * 8 * _GRP,), jnp.float32),
        pltpu.SemaphoreType.DMA,
    ],
    compiler_params=pltpu.CompilerParams(
        use_tc_tiling_on_sc=False, needs_layout_passes=False
    ),
)
def _sc_gather(table_hbm, idx_hbm, out_hbm, idx_v, rows_v, tb_v, sem):
    wid = lax.axis_index("s") * _NC + lax.axis_index("c")

    def body(c, carry):
        m = wid * _NCH + c
        s = m // (_NB // _CH)
        bq = m % (_NB // _CH)
        gbase = s * (_NB // _GRP) + bq * _GPC
        pltpu.sync_copy(idx_hbm.at[pl.ds(gbase, _GPC)], idx_v)
        copies = [
            pltpu.async_copy(
                table_hbm.at[idx_v.at[j]],
                rows_v.at[pl.ds(j * _GRP, _GRP)],
                sem,
            )
            for j in range(_GPC)
        ]
        for cp in copies:
            cp.wait()

        lane = lax.iota(jnp.int32, 16)

        @plsc.parallel_loop(0, _GPC * 8, unroll=8)
        def tr_body(jv):
            jt = jv // 8
            v = jv % 8
            row0 = jt * _GRP + v * 16
            base = jt * 1024 + v * 16
            rv16 = rows_v.at[pl.ds(row0, 16)]
            for k in range(DIM // 8):
                for dd in range(8):
                    col = jnp.full((16,), k * 8 + dd, jnp.int32)
                    vals = plsc.load_gather(rv16, [lane, col])
                    tb_v[pl.ds(base + k * 8192 + dd * 128, 16)] = vals

        for k in range(DIM // 8):
            pltpu.sync_copy(
                tb_v.at[pl.ds(k * 8192, 8192)],
                out_hbm.at[s, k, pl.ds(bq * 8192, 8192)],
            )
        return carry

    lax.fori_loop(0, _NCH, body, 0)


def kernel(input_ids, mu, rho):
    mu_t = mu.T
    rho_t = rho.T
    w4, klp = _tc_pass(mu_t, rho_t, _EPS_T)
    table = w4.reshape(NUM_EMB, DIM)
    idx2 = input_ids.T.reshape(_B // _GRP, _GRP)
    out3 = _sc_gather(table, idx2)
    out5 = out3.reshape(_NS, DIM // 8, _NB // 128, 8, 128)
    after_embed = out5.transpose(2, 4, 0, 1, 3).reshape(_NB, _NS, DIM)
    return after_embed, klp[0, 0]


# speedup vs baseline: 1.1449x; 1.1449x over previous
import functools
import math

import jax
import jax.numpy as jnp
from jax import lax
from jax.experimental import pallas as pl
from jax.experimental.pallas import tpu as pltpu
from jax.experimental.pallas import tpu_sc as plsc

NUM_EMB = 1000000
DIM = 32
PI = 0.25
S1 = 1.0
S2 = math.exp(-6.0)

_C0 = -0.5 * math.log(2.0 * math.pi)
_K1 = math.log(PI) + _C0 - math.log(S1)
_K3_1 = 1.0 / (2.0 * S1 * S1)
_K2 = math.log(1.0 - PI) + _C0 - math.log(S2)
_K3_2 = 1.0 / (2.0 * S2 * S2)
_KL_CONST = float(NUM_EMB * DIM * _C0)

import numpy as _np


def _np_threefry2x32(k0, k1, x0, x1):
    def rotl(x, r):
        return ((x << _np.uint32(r)) | (x >> _np.uint32(32 - r))).astype(_np.uint32)

    ks0 = _np.uint32(k0)
    ks1 = _np.uint32(k1)
    ks2 = _np.uint32(ks0 ^ ks1 ^ _np.uint32(0x1BD11BDA))
    ks = [ks0, ks1, ks2]
    rots = [[13, 15, 26, 6], [17, 29, 16, 24]]
    x0 = (x0 + ks0).astype(_np.uint32)
    x1 = (x1 + ks1).astype(_np.uint32)
    for i in range(5):
        for r in rots[i % 2]:
            x0 = (x0 + x1).astype(_np.uint32)
            x1 = rotl(x1, r)
            x1 = (x1 ^ x0).astype(_np.uint32)
        x0 = (x0 + ks[(i + 1) % 3]).astype(_np.uint32)
        x1 = (x1 + ks[(i + 2) % 3] + _np.uint32(i + 1)).astype(_np.uint32)
    return x0, x1


def _np_erfinv32(x):
    x = x.astype(_np.float32)
    w = (-_np.log1p((-x * x).astype(_np.float32))).astype(_np.float32)
    small = w < _np.float32(5.0)
    ws = (w - _np.float32(2.5)).astype(_np.float32)
    wl = (_np.sqrt(_np.maximum(w, _np.float32(5.0))) - _np.float32(3.0)).astype(
        _np.float32
    )
    cs = [2.81022636e-08, 3.43273939e-07, -3.5233877e-06, -4.39150654e-06,
          0.00021858087, -0.00125372503, -0.00417768164, 0.246640727, 1.50140941]
    cl = [-0.000200214257, 0.000100950558, 0.00134934322, -0.00367342844,
          0.00573950773, -0.0076224613, 0.00943887047, 1.00167406, 2.83297682]
    ps = _np.float32(cs[0])
    for c in cs[1:]:
        ps = (_np.float32(c) + ps * ws).astype(_np.float32)
    pw = _np.float32(cl[0])
    for c in cl[1:]:
        pw = (_np.float32(c) + pw * wl).astype(_np.float32)
    p = _np.where(small, ps, pw)
    return (p * x).astype(_np.float32)


def _np_normal_key42(n):
    i = _np.arange(n, dtype=_np.uint64)
    b0, b1 = _np_threefry2x32(
        0, 42,
        (i >> _np.uint64(32)).astype(_np.uint32),
        (i & _np.uint64(0xFFFFFFFF)).astype(_np.uint32),
    )
    bits = (b0 ^ b1).astype(_np.uint32)
    f = ((bits >> _np.uint32(9)) | _np.uint32(0x3F800000)).view(_np.float32)
    u01 = (f - _np.float32(1.0)).astype(_np.float32)
    lo = _np.nextafter(_np.float32(-1.0), _np.float32(0.0), dtype=_np.float32)
    hi = _np.float32(1.0)
    u = _np.maximum(lo, (u01 * (hi - lo) + lo).astype(_np.float32))
    return (_np.float32(_np.sqrt(2.0)) * _np_erfinv32(u)).astype(_np.float32)


_EPS_T = _np.ascontiguousarray(
    _np_normal_key42(NUM_EMB * DIM).reshape(NUM_EMB, DIM).T
)

_BLKC = 8192
_GRID = -(-NUM_EMB // _BLKC)


def _tc_body_w(mu_ref, rho_ref, eps_ref, w_ref):
    mu = mu_ref[...]
    rho = rho_ref[...]
    eps = eps_ref[...]
    sig = jnp.log1p(jnp.exp(rho)) + 1e-5
    w = mu + eps * sig
    wt = w.T.reshape(_BLKC // 4, 4, DIM)
    w_ref[...] = jnp.concatenate([wt[:, 0], wt[:, 1], wt[:, 2], wt[:, 3]], axis=1)


_tc_pass_w = pl.pallas_call(
    _tc_body_w,
    grid=(_GRID,),
    in_specs=[
        pl.BlockSpec((DIM, _BLKC), lambda i: (0, i)),
        pl.BlockSpec((DIM, _BLKC), lambda i: (0, i)),
        pl.BlockSpec((DIM, _BLKC), lambda i: (0, i)),
    ],
    out_specs=pl.BlockSpec((_BLKC // 4, 128), lambda i: (i, 0)),
    out_shape=jax.ShapeDtypeStruct((NUM_EMB * DIM // 128, 128), jnp.float32),
    compiler_params=pltpu.CompilerParams(
        dimension_semantics=("arbitrary",),
    ),
)


def _tc_body_kl(mu_ref, rho_ref, eps_ref, kl_ref, acc_ref):
    i = pl.program_id(0)
    mu = mu_ref[...]
    rho = rho_ref[...]
    eps = eps_ref[...]
    sig = jnp.log1p(jnp.exp(rho)) + 1e-5
    w = mu + eps * sig
    t = w * w
    lp1 = _K1 - _K3_1 * t
    lp2 = _K2 - _K3_2 * t
    m = jnp.maximum(lp1, lp2)
    log_prior = m + jnp.log1p(jnp.exp(-jnp.abs(lp1 - lp2)))
    term = -jnp.log(sig) - 0.5 * (eps * eps) - log_prior
    col = i * _BLKC + lax.broadcasted_iota(jnp.int32, (DIM, _BLKC), 1)
    term = jnp.where(col < NUM_EMB, term, 0.0)
    part = jnp.sum(term.reshape(4, 8, _BLKC), axis=0)

    @pl.when(i == 0)
    def _init():
        acc_ref[...] = part

    @pl.when(i != 0)
    def _acc():
        acc_ref[...] = acc_ref[...] + part

    @pl.when(i == _GRID - 1)
    def _fin():
        kl_ref[0, 0] = _KL_CONST + jnp.sum(acc_ref[...])


_tc_pass_kl = pl.pallas_call(
    _tc_body_kl,
    grid=(_GRID,),
    in_specs=[
        pl.BlockSpec((DIM, _BLKC), lambda i: (0, i)),
        pl.BlockSpec((DIM, _BLKC), lambda i: (0, i)),
        pl.BlockSpec((DIM, _BLKC), lambda i: (0, i)),
    ],
    out_specs=pl.BlockSpec((1, 1), lambda i: (0, 0), memory_space=pltpu.SMEM),
    out_shape=jax.ShapeDtypeStruct((1, 1), jnp.float32),
    scratch_shapes=[pltpu.VMEM((8, _BLKC), jnp.float32)],
    compiler_params=pltpu.CompilerParams(
        dimension_semantics=("arbitrary",),
    ),
)

_INFO = plsc.get_sparse_core_info()
_NC = _INFO.num_cores
_NW = _INFO.num_cores * _INFO.num_subcores
_B = 16384 * 50
_NB = 16384
_NS = 50
_GRP = 128
_GPC = 8
_CH = _GRP * _GPC
_NCHT = _NS * (_NB // _CH)
_NCH = _NCHT // _NW

_sc_mesh = plsc.VectorSubcoreMesh(core_axis_name="c", subcore_axis_name="s")


@functools.partial(
    pl.kernel,
    mesh=_sc_mesh,
    out_type=jax.ShapeDtypeStruct((_NS, DIM // 8, (_NB // 128) * 8 * 128), jnp.float32),
    scratch_types=[
        pltpu.VMEM((_GPC, _GRP), jnp.int32),
        pltpu.VMEM((_CH, DIM), jnp.float32),
        pltpu.VMEM(((DIM // 8) * _GPC * 8 * _GRP,), jnp.float32),
        pltpu.SemaphoreType.DMA,
    ],
    compiler_params=pltpu.CompilerParams(
        use_tc_tiling_on_sc=False, needs_layout_passes=False
    ),
)
def _sc_gather(table_hbm, idx_hbm, out_hbm, idx_v, rows_v, tb_v, sem):
    wid = lax.axis_index("s") * _NC + lax.axis_index("c")

    def body(c, carry):
        m = wid * _NCH + c
        s = m // (_NB // _CH)
        bq = m % (_NB // _CH)
        gbase = s * (_NB // _GRP) + bq * _GPC
        pltpu.sync_copy(idx_hbm.at[pl.ds(gbase, _GPC)], idx_v)
        copies = [
            pltpu.async_copy(
                table_hbm.at[idx_v.at[j]],
                rows_v.at[pl.ds(j * _GRP, _GRP)],
                sem,
            )
            for j in range(_GPC)
        ]
        for cp in copies:
            cp.wait()

        lane = lax.iota(jnp.int32, 16)

        @plsc.parallel_loop(0, _GPC * 8, unroll=4)
        def tr_body(jv):
            jt = jv // 8
            v = jv % 8
            row0 = jt * _GRP + v * 16
            base = jt * 1024 + v * 16
            rv16 = rows_v.at[pl.ds(row0, 16)]
            for k in range(DIM // 8):
                for dd in range(8):
                    col = jnp.full((16,), k * 8 + dd, jnp.int32)
                    vals = plsc.load_gather(rv16, [lane, col])
                    tb_v[pl.ds(base + k * 8192 + dd * 128, 16)] = vals

        for k in range(DIM // 8):
            pltpu.sync_copy(
                tb_v.at[pl.ds(k * 8192, 8192)],
                out_hbm.at[s, k, pl.ds(bq * 8192, 8192)],
            )
        return carry

    lax.fori_loop(0, _NCH, body, 0)


def kernel(input_ids, mu, rho):
    mu_t = mu.T
    rho_t = rho.T
    w4 = _tc_pass_w(mu_t, rho_t, _EPS_T)
    klp = _tc_pass_kl(mu_t, rho_t, _EPS_T)
    table = w4.reshape(NUM_EMB, DIM)
    idx2 = input_ids.T.reshape(_B // _GRP, _GRP)
    out3 = _sc_gather(table, idx2)
    out5 = out3.reshape(_NS, DIM // 8, _NB // 128, 8, 128)
    after_embed = out5.transpose(2, 4, 0, 1, 3).reshape(_NB, _NS, DIM)
    return after_embed, klp[0, 0]


# speedup vs baseline: 1.2157x; 1.0618x over previous
import functools
import math

import jax
import jax.numpy as jnp
from jax import lax
from jax.experimental import pallas as pl
from jax.experimental.pallas import tpu as pltpu
from jax.experimental.pallas import tpu_sc as plsc

NUM_EMB = 1000000
DIM = 32
PI = 0.25
S1 = 1.0
S2 = math.exp(-6.0)

_C0 = -0.5 * math.log(2.0 * math.pi)
_K1 = math.log(PI) + _C0 - math.log(S1)
_K3_1 = 1.0 / (2.0 * S1 * S1)
_K2 = math.log(1.0 - PI) + _C0 - math.log(S2)
_K3_2 = 1.0 / (2.0 * S2 * S2)
_KL_CONST = float(NUM_EMB * DIM * _C0)

import numpy as _np


def _np_threefry2x32(k0, k1, x0, x1):
    def rotl(x, r):
        return ((x << _np.uint32(r)) | (x >> _np.uint32(32 - r))).astype(_np.uint32)

    ks0 = _np.uint32(k0)
    ks1 = _np.uint32(k1)
    ks2 = _np.uint32(ks0 ^ ks1 ^ _np.uint32(0x1BD11BDA))
    ks = [ks0, ks1, ks2]
    rots = [[13, 15, 26, 6], [17, 29, 16, 24]]
    x0 = (x0 + ks0).astype(_np.uint32)
    x1 = (x1 + ks1).astype(_np.uint32)
    for i in range(5):
        for r in rots[i % 2]:
            x0 = (x0 + x1).astype(_np.uint32)
            x1 = rotl(x1, r)
            x1 = (x1 ^ x0).astype(_np.uint32)
        x0 = (x0 + ks[(i + 1) % 3]).astype(_np.uint32)
        x1 = (x1 + ks[(i + 2) % 3] + _np.uint32(i + 1)).astype(_np.uint32)
    return x0, x1


def _np_erfinv32(x):
    x = x.astype(_np.float32)
    w = (-_np.log1p((-x * x).astype(_np.float32))).astype(_np.float32)
    small = w < _np.float32(5.0)
    ws = (w - _np.float32(2.5)).astype(_np.float32)
    wl = (_np.sqrt(_np.maximum(w, _np.float32(5.0))) - _np.float32(3.0)).astype(
        _np.float32
    )
    cs = [2.81022636e-08, 3.43273939e-07, -3.5233877e-06, -4.39150654e-06,
          0.00021858087, -0.00125372503, -0.00417768164, 0.246640727, 1.50140941]
    cl = [-0.000200214257, 0.000100950558, 0.00134934322, -0.00367342844,
          0.00573950773, -0.0076224613, 0.00943887047, 1.00167406, 2.83297682]
    ps = _np.float32(cs[0])
    for c in cs[1:]:
        ps = (_np.float32(c) + ps * ws).astype(_np.float32)
    pw = _np.float32(cl[0])
    for c in cl[1:]:
        pw = (_np.float32(c) + pw * wl).astype(_np.float32)
    p = _np.where(small, ps, pw)
    return (p * x).astype(_np.float32)


def _np_normal_key42(n):
    i = _np.arange(n, dtype=_np.uint64)
    b0, b1 = _np_threefry2x32(
        0, 42,
        (i >> _np.uint64(32)).astype(_np.uint32),
        (i & _np.uint64(0xFFFFFFFF)).astype(_np.uint32),
    )
    bits = (b0 ^ b1).astype(_np.uint32)
    f = ((bits >> _np.uint32(9)) | _np.uint32(0x3F800000)).view(_np.float32)
    u01 = (f - _np.float32(1.0)).astype(_np.float32)
    lo = _np.nextafter(_np.float32(-1.0), _np.float32(0.0), dtype=_np.float32)
    hi = _np.float32(1.0)
    u = _np.maximum(lo, (u01 * (hi - lo) + lo).astype(_np.float32))
    return (_np.float32(_np.sqrt(2.0)) * _np_erfinv32(u)).astype(_np.float32)


_EPS_T = _np.ascontiguousarray(
    _np_normal_key42(NUM_EMB * DIM).reshape(NUM_EMB, DIM).T
)

_BLKC = 8192
_GRID = -(-NUM_EMB // _BLKC)


def _tc_body_w(mu_ref, rho_ref, eps_ref, w_ref):
    mu = mu_ref[...]
    rho = rho_ref[...]
    eps = eps_ref[...]
    sig = jnp.log1p(jnp.exp(rho)) + 1e-5
    w = mu + eps * sig
    wt = w.T.reshape(_BLKC // 4, 4, DIM)
    w_ref[...] = jnp.concatenate([wt[:, 0], wt[:, 1], wt[:, 2], wt[:, 3]], axis=1)


_tc_pass_w = pl.pallas_call(
    _tc_body_w,
    grid=(_GRID,),
    in_specs=[
        pl.BlockSpec((DIM, _BLKC), lambda i: (0, i)),
        pl.BlockSpec((DIM, _BLKC), lambda i: (0, i)),
        pl.BlockSpec((DIM, _BLKC), lambda i: (0, i)),
    ],
    out_specs=pl.BlockSpec((_BLKC // 4, 128), lambda i: (i, 0)),
    out_shape=jax.ShapeDtypeStruct((NUM_EMB * DIM // 128, 128), jnp.float32),
    compiler_params=pltpu.CompilerParams(
        dimension_semantics=("arbitrary",),
    ),
)


def _tc_body_kl(mu_ref, rho_ref, eps_ref, kl_ref, acc_ref):
    i = pl.program_id(0)
    mu = mu_ref[...]
    rho = rho_ref[...]
    eps = eps_ref[...]
    sig = jnp.log1p(jnp.exp(rho)) + 1e-5
    w = mu + eps * sig
    t = w * w
    lp1 = _K1 - _K3_1 * t
    lp2 = _K2 - _K3_2 * t
    m = jnp.maximum(lp1, lp2)
    log_prior = m + jnp.log1p(jnp.exp(-jnp.abs(lp1 - lp2)))
    term = -jnp.log(sig) - 0.5 * (eps * eps) - log_prior
    col = i * _BLKC + lax.broadcasted_iota(jnp.int32, (DIM, _BLKC), 1)
    term = jnp.where(col < NUM_EMB, term, 0.0)
    part = jnp.sum(term.reshape(4, 8, _BLKC), axis=0)

    @pl.when(i == 0)
    def _init():
        acc_ref[...] = part

    @pl.when(i != 0)
    def _acc():
        acc_ref[...] = acc_ref[...] + part

    @pl.when(i == _GRID - 1)
    def _fin():
        kl_ref[0, 0] = _KL_CONST + jnp.sum(acc_ref[...])


_tc_pass_kl = pl.pallas_call(
    _tc_body_kl,
    grid=(_GRID,),
    in_specs=[
        pl.BlockSpec((DIM, _BLKC), lambda i: (0, i)),
        pl.BlockSpec((DIM, _BLKC), lambda i: (0, i)),
        pl.BlockSpec((DIM, _BLKC), lambda i: (0, i)),
    ],
    out_specs=pl.BlockSpec((1, 1), lambda i: (0, 0), memory_space=pltpu.SMEM),
    out_shape=jax.ShapeDtypeStruct((1, 1), jnp.float32),
    scratch_shapes=[pltpu.VMEM((8, _BLKC), jnp.float32)],
    compiler_params=pltpu.CompilerParams(
        dimension_semantics=("arbitrary",),
    ),
)

_INFO = plsc.get_sparse_core_info()
_NC = _INFO.num_cores
_NW = _INFO.num_cores * _INFO.num_subcores
_B = 16384 * 50
_NB = 16384
_NS = 50
_GRP = 128
_GPC = 8
_CH = _GRP * _GPC
_NCHT = _NS * (_NB // _CH)
_NCH = _NCHT // _NW

_sc_mesh = plsc.VectorSubcoreMesh(core_axis_name="c", subcore_axis_name="s")


@functools.partial(
    pl.kernel,
    mesh=_sc_mesh,
    out_type=jax.ShapeDtypeStruct((_NS, DIM // 8, (_NB // 128) * 8 * 128), jnp.float32),
    scratch_types=[
        pltpu.VMEM((_GPC, _GRP), jnp.int32),
        pltpu.VMEM((_GPC, _GRP), jnp.int32),
        pltpu.VMEM((_CH, DIM), jnp.float32),
        pltpu.VMEM((_CH, DIM), jnp.float32),
        pltpu.VMEM(((DIM // 8) * _GPC * 8 * _GRP,), jnp.float32),
        pltpu.SemaphoreType.DMA,
        pltpu.SemaphoreType.DMA,
    ],
    compiler_params=pltpu.CompilerParams(
        use_tc_tiling_on_sc=False, needs_layout_passes=False
    ),
)
def _sc_gather(
    table_hbm, idx_hbm, out_hbm, idx_v0, idx_v1, rows_v0, rows_v1, tb_v, sem0, sem1
):
    wid = lax.axis_index("s") * _NC + lax.axis_index("c")
    lane = lax.iota(jnp.int32, 16)

    def fire(c, idx_v, rows_v, sem):
        m = wid * _NCH + c
        s = m // (_NB // _CH)
        bq = m % (_NB // _CH)
        gbase = s * (_NB // _GRP) + bq * _GPC
        pltpu.sync_copy(idx_hbm.at[pl.ds(gbase, _GPC)], idx_v)
        for j in range(_GPC):
            pltpu.async_copy(
                table_hbm.at[idx_v.at[j]],
                rows_v.at[pl.ds(j * _GRP, _GRP)],
                sem,
            )

    def finish(c, rows_v, sem):
        m = wid * _NCH + c
        s = m // (_NB // _CH)
        bq = m % (_NB // _CH)
        for j in range(_GPC):
            pltpu.make_async_copy(
                table_hbm.at[idx_v0.at[j]],
                rows_v.at[pl.ds(j * _GRP, _GRP)],
                sem,
            ).wait()

        @plsc.parallel_loop(0, _GPC * 8, unroll=4)
        def tr_body(jv):
            jt = jv // 8
            v = jv % 8
            row0 = jt * _GRP + v * 16
            base = jt * 1024 + v * 16
            rv16 = rows_v.at[pl.ds(row0, 16)]
            for k in range(DIM // 8):
                for dd in range(8):
                    col = jnp.full((16,), k * 8 + dd, jnp.int32)
                    vals = plsc.load_gather(rv16, [lane, col])
                    tb_v[pl.ds(base + k * 8192 + dd * 128, 16)] = vals

        for k in range(DIM // 8):
            pltpu.sync_copy(
                tb_v.at[pl.ds(k * 8192, 8192)],
                out_hbm.at[s, k, pl.ds(bq * 8192, 8192)],
            )

    fire(0, idx_v0, rows_v0, sem0)

    def body(h, carry):
        c0 = 2 * h
        fire(c0 + 1, idx_v1, rows_v1, sem1)
        finish(c0, rows_v0, sem0)
        fire(c0 + 2, idx_v0, rows_v0, sem0)
        finish(c0 + 1, rows_v1, sem1)
        return carry

    lax.fori_loop(0, (_NCH - 1) // 2, body, 0)
    finish(_NCH - 1, rows_v0, sem0)


def kernel(input_ids, mu, rho):
    mu_t = mu.T
    rho_t = rho.T
    w4 = _tc_pass_w(mu_t, rho_t, _EPS_T)
    klp = _tc_pass_kl(mu_t, rho_t, _EPS_T)
    table = w4.reshape(NUM_EMB, DIM)
    idx2 = input_ids.T.reshape(_B // _GRP, _GRP)
    out3 = _sc_gather(table, idx2)
    out5 = out3.reshape(_NS, DIM // 8, _NB // 128, 8, 128)
    after_embed = out5.transpose(2, 4, 0, 1, 3).reshape(_NB, _NS, DIM)
    return after_embed, klp[0, 0]
